# baseline (speedup 1.0000x reference)
import jax
import jax.numpy as jnp
from jax.experimental import pallas as pl
from jax.experimental.pallas import tpu as pltpu

_B = 16
_N = 2048
_S1 = 410
_S1P = 416
_S2 = 103
_S2P = 104
_R1 = 0.2
_R2 = 0.4
_K = 64
_BIG = 1e30
_PADC = 1e5


def _fps_kernel(posL_ref, pos1L_ref, pos2L_ref, d1_ref, d2_ref):
    pos1L_ref[...] = jnp.full((3, _S1P, _B), _PADC, jnp.float32)
    pos2L_ref[...] = jnp.full((3, _S2P, _B), _PADC, jnp.float32)
    d0 = jnp.zeros((_N, _B), jnp.float32)
    for l in range(3):
        pos1L_ref[l, 0, :] = posL_ref[l, 0, :]
        df = posL_ref[l] - posL_ref[l, 0:1, :]
        d0 = d0 + df * df
    d1_ref[...] = d0

    def body1(i, carry):
        d = d1_ref[...]
        nxt = jnp.argmax(d, axis=0)
        onehot = jax.lax.broadcasted_iota(jnp.int32, (_N, _B), 0) == nxt[None, :]
        dn = jnp.zeros((_N, _B), jnp.float32)
        for l in range(3):
            c = jnp.max(jnp.where(onehot, posL_ref[l], -1.0), axis=0)
            pos1L_ref[l, i, :] = c
            df = posL_ref[l] - c[None, :]
            dn = dn + df * df
        d1_ref[...] = jnp.minimum(d, dn)
        return carry

    jax.lax.fori_loop(1, _S1, body1, 0)

    d0b = jnp.zeros((_S1P, _B), jnp.float32)
    for l in range(3):
        pos2L_ref[l, 0, :] = pos1L_ref[l, 0, :]
        df = pos1L_ref[l] - pos1L_ref[l, 0:1, :]
        d0b = d0b + df * df
    rows = jax.lax.broadcasted_iota(jnp.int32, (_S1P, _B), 0)
    d2_ref[...] = jnp.where(rows < _S1, d0b, -_BIG)

    def body2(i, carry):
        d = d2_ref[...]
        nxt = jnp.argmax(d, axis=0)
        onehot = jax.lax.broadcasted_iota(jnp.int32, (_S1P, _B), 0) == nxt[None, :]
        dn = jnp.zeros((_S1P, _B), jnp.float32)
        for l in range(3):
            c = jnp.max(jnp.where(onehot, pos1L_ref[l], -1.0), axis=0)
            pos2L_ref[l, i, :] = c
            df = pos1L_ref[l] - c[None, :]
            dn = dn + df * df
        d2_ref[...] = jnp.minimum(d, jnp.where(rows < _S1, dn, -_BIG))
        return carry

    jax.lax.fori_loop(1, _S2, body2, 0)


def _sa1_kernel(pos1_ref, posT_ref, w1_ref, b1_ref, w2_ref, b2_ref,
                w3_ref, b3_ref, x1_ref, d2_ref):
    pos1 = pos1_ref[0]
    posT = posT_ref[0]
    d2 = jnp.zeros((_S1P, _N), jnp.float32)
    for l in range(3):
        df = pos1[:, l:l + 1] - posT[l:l + 1, :]
        d2 = d2 + df * df
    d2_ref[...] = jnp.where(d2 <= _R1 * _R1, d2, _BIG)
    x1_ref[0] = jnp.full((_S1P, 128), -1.0, jnp.float32)

    dot = lambda a, b: jnp.dot(a, b, preferred_element_type=jnp.float32,
                               precision=jax.lax.Precision.HIGHEST)

    def body(i, carry):
        d = d2_ref[...]
        idx = jnp.argmin(d, axis=1)
        m2 = jnp.min(d, axis=1, keepdims=True)
        onehot = jax.lax.broadcasted_iota(jnp.int32, (_S1P, _N), 1) == idx[:, None]
        d2_ref[...] = jnp.where(onehot, _BIG, d)
        cols = []
        for l in range(3):
            c = jnp.max(jnp.where(onehot, posT[l:l + 1, :], -1.0), axis=1)
            cols.append(c[:, None])
        cols.append(jnp.zeros((_S1P, 5), jnp.float32))
        rel = jnp.concatenate(cols, axis=1) - pos1
        h = jax.nn.relu(dot(rel, w1_ref[...]) + b1_ref[...])
        h = jax.nn.relu(dot(h, w2_ref[...]) + b2_ref[...])
        h = jax.nn.relu(dot(h, w3_ref[...]) + b3_ref[...])
        h = jnp.where(m2 < _BIG * 0.5, h, -1.0)
        x1_ref[0] = jnp.maximum(x1_ref[0], h)
        return carry

    jax.lax.fori_loop(0, _K, body, 0)


def _sa2_kernel(pos2_ref, pos1T_ref, x1_ref, w1a_ref, w1b_ref, b1_ref,
                w2_ref, b2_ref, w3_ref, b3_ref, x2_ref, d2_ref):
    pos2 = pos2_ref[0]
    pos1T = pos1T_ref[0]
    x1 = x1_ref[0]
    d2 = jnp.zeros((_S2P, _S1P), jnp.float32)
    for l in range(3):
        df = pos2[:, l:l + 1] - pos1T[l:l + 1, :]
        d2 = d2 + df * df
    d2_ref[...] = jnp.where(d2 <= _R2 * _R2, d2, _BIG)
    x2_ref[0] = jnp.full((_S2P, 256), -1.0, jnp.float32)

    dot = lambda a, b: jnp.dot(a, b, preferred_element_type=jnp.float32,
                               precision=jax.lax.Precision.HIGHEST)

    def body(i, carry):
        d = d2_ref[...]
        idx = jnp.argmin(d, axis=1)
        m2 = jnp.min(d, axis=1, keepdims=True)
        onehot = jax.lax.broadcasted_iota(jnp.int32, (_S2P, _S1P), 1) == idx[:, None]
        d2_ref[...] = jnp.where(onehot, _BIG, d)
        cols = []
        for l in range(3):
            c = jnp.max(jnp.where(onehot, pos1T[l:l + 1, :], -1.0), axis=1)
            cols.append(c[:, None])
        cols.append(jnp.zeros((_S2P, 5), jnp.float32))
        rel = jnp.concatenate(cols, axis=1) - pos2
        xg = dot(onehot.astype(jnp.float32), x1)
        h = jax.nn.relu(dot(xg, w1a_ref[...]) + dot(rel, w1b_ref[...]) + b1_ref[...])
        h = jax.nn.relu(dot(h, w2_ref[...]) + b2_ref[...])
        h = jax.nn.relu(dot(h, w3_ref[...]) + b3_ref[...])
        h = jnp.where(m2 < _BIG * 0.5, h, -1.0)
        x2_ref[0] = jnp.maximum(x2_ref[0], h)
        return carry

    jax.lax.fori_loop(0, _K, body, 0)


def _knn_wmat(dst, srcT, nd, ns):
    d2 = jnp.zeros((nd, ns), jnp.float32)
    for l in range(3):
        df = dst[:, l:l + 1] - srcT[l:l + 1, :]
        d2 = d2 + df * df
    wm = jnp.zeros((nd, ns), jnp.float32)
    wsum = jnp.zeros((nd, 1), jnp.float32)
    for _ in range(3):
        idx = jnp.argmin(d2, axis=1)
        m = jnp.min(d2, axis=1)
        onehot = jax.lax.broadcasted_iota(jnp.int32, (nd, ns), 1) == idx[:, None]
        w = 1.0 / jnp.maximum(m, 1e-16)
        wm = wm + jnp.where(onehot, w[:, None], 0.0)
        wsum = wsum + w[:, None]
        d2 = jnp.where(onehot, _BIG, d2)
    return wm / wsum


def _tail_kernel(x2_ref, pos2_ref, pos2T_ref, pos1_ref, pos1T_ref, x1_ref,
                 pos_ref, ts_ref,
                 tw_ref, tb_ref,
                 g1a_ref, g1b_ref, gb1_ref, g2_ref, gb2_ref, g3_ref, gb3_ref,
                 f1a_ref, f1b_ref, fb1_ref, f2_ref, fb2_ref,
                 a1x_ref, a1y_ref, ab1_ref, a2_ref, ab2_ref,
                 p1_ref, pb1_ref, p2_ref, pb2_ref, p3_ref, pb3_ref,
                 l1_ref, lb1_ref, l2_ref, lb2_ref, l3_ref, lb3_ref,
                 out_ref):
    x2 = x2_ref[0]
    pos2 = pos2_ref[0]
    pos2T = pos2T_ref[0]
    pos1 = pos1_ref[0]
    pos1T = pos1T_ref[0]
    x1 = x1_ref[0]
    posb = pos_ref[0]

    dot = lambda a, b: jnp.dot(a, b, preferred_element_type=jnp.float32,
                               precision=jax.lax.Precision.HIGHEST)

    t = jax.nn.relu(dot(ts_ref[0], tw_ref[...]) + tb_ref[...])

    h = jax.nn.relu(dot(x2, g1a_ref[...]) + dot(pos2, g1b_ref[...]) + gb1_ref[...])
    h = jax.nn.relu(dot(h, g2_ref[...]) + gb2_ref[...])
    h = jax.nn.relu(dot(h, g3_ref[...]) + gb3_ref[...])
    rows = jax.lax.broadcasted_iota(jnp.int32, (_S2P, 1024), 0)
    h = jnp.where(rows < _S2, h, -_BIG)
    x3 = jnp.max(h, axis=0, keepdims=True)
    t8 = jnp.concatenate([t] * 8, axis=1)
    x3 = x3 * t8

    fh = jax.nn.relu(dot(x3, f1a_ref[...]) + dot(x2, f1b_ref[...]) + fb1_ref[...])
    fp3_x = jax.nn.relu(dot(fh, f2_ref[...]) + fb2_ref[...])
    t2 = jnp.concatenate([t] * 2, axis=1)
    fp3_x = fp3_x * t2

    wm2 = _knn_wmat(pos1, pos2T, _S1P, _S2P)
    fp2_i = dot(wm2, fp3_x)
    fh = jax.nn.relu(dot(fp2_i, a1x_ref[...]) + dot(x1, a1y_ref[...]) + ab1_ref[...])
    fp2_x = jax.nn.relu(dot(fh, a2_ref[...]) + ab2_ref[...])
    fp2_x = fp2_x * t

    wm1 = _knn_wmat(posb, pos1T, _N, _S1P)
    fp1_i = dot(wm1, fp2_x)
    fh = jax.nn.relu(dot(fp1_i, p1_ref[...]) + pb1_ref[...])
    fh = jax.nn.relu(dot(fh, p2_ref[...]) + pb2_ref[...])
    fh = jax.nn.relu(dot(fh, p3_ref[...]) + pb3_ref[...])

    fh = jax.nn.relu(dot(fh, l1_ref[...]) + lb1_ref[...])
    fh = jax.nn.relu(dot(fh, l2_ref[...]) + lb2_ref[...])
    out_ref[0] = dot(fh, l3_ref[...]) + lb3_ref[...]


def _b2(b):
    return b.reshape(1, -1)


def kernel(pos, batch, task_sp, params):
    f32 = jnp.float32
    posb = pos.reshape(_B, _N, 3).astype(f32)
    pos8 = jnp.concatenate([posb, jnp.zeros((_B, _N, 5), f32)], axis=2)
    posT = jnp.concatenate([jnp.transpose(posb, (0, 2, 1)),
                            jnp.zeros((_B, 5, _N), f32)], axis=1)
    posL = jnp.transpose(posb, (2, 1, 0))

    pos1L, pos2L = pl.pallas_call(
        _fps_kernel,
        out_shape=[jax.ShapeDtypeStruct((3, _S1P, _B), f32),
                   jax.ShapeDtypeStruct((3, _S2P, _B), f32)],
        scratch_shapes=[pltpu.VMEM((_N, _B), f32),
                        pltpu.VMEM((_S1P, _B), f32)],
    )(posL)

    pos1b = jnp.transpose(pos1L, (2, 1, 0))
    pos2b = jnp.transpose(pos2L, (2, 1, 0))
    pos1 = jnp.concatenate([pos1b, jnp.zeros((_B, _S1P, 5), f32)], axis=2)
    pos2 = jnp.concatenate([pos2b, jnp.zeros((_B, _S2P, 5), f32)], axis=2)
    pos1T = jnp.concatenate([jnp.transpose(pos1b, (0, 2, 1)),
                             jnp.zeros((_B, 5, _S1P), f32)], axis=1)
    pos2T = jnp.concatenate([jnp.transpose(pos2b, (0, 2, 1)),
                             jnp.zeros((_B, 5, _S2P), f32)], axis=1)

    (w1, b1), (w2, b2), (w3, b3) = params["sa1"]
    w1p = jnp.concatenate([w1, jnp.zeros((5, 64), f32)], axis=0)

    def bspec(shape):
        return pl.BlockSpec((1,) + shape, lambda i: (i, 0, 0))

    def wspec2(a):
        return pl.BlockSpec(a.shape, lambda i: (0,) * a.ndim)

    x1 = pl.pallas_call(
        _sa1_kernel,
        grid=(_B,),
        compiler_params=pltpu.CompilerParams(dimension_semantics=("parallel",)),
        in_specs=[bspec((_S1P, 8)), bspec((8, _N)),
                  wspec2(w1p), wspec2(_b2(b1)), wspec2(w2), wspec2(_b2(b2)),
                  wspec2(w3), wspec2(_b2(b3))],
        out_specs=bspec((_S1P, 128)),
        out_shape=jax.ShapeDtypeStruct((_B, _S1P, 128), f32),
        scratch_shapes=[pltpu.VMEM((_S1P, _N), f32)],
    )(pos1, posT, w1p, _b2(b1), w2, _b2(b2), w3, _b2(b3))

    (w1, b1), (w2, b2), (w3, b3) = params["sa2"]
    w1a = w1[:128]
    w1b = jnp.concatenate([w1[128:131], jnp.zeros((5, 128), f32)], axis=0)
    x2 = pl.pallas_call(
        _sa2_kernel,
        grid=(_B,),
        compiler_params=pltpu.CompilerParams(dimension_semantics=("parallel",)),
        in_specs=[bspec((_S2P, 8)), bspec((8, _S1P)), bspec((_S1P, 128)),
                  wspec2(w1a), wspec2(w1b), wspec2(_b2(b1)),
                  wspec2(w2), wspec2(_b2(b2)), wspec2(w3), wspec2(_b2(b3))],
        out_specs=bspec((_S2P, 256)),
        out_shape=jax.ShapeDtypeStruct((_B, _S2P, 256), f32),
        scratch_shapes=[pltpu.VMEM((_S2P, _S1P), f32)],
    )(pos2, pos1T, x1, w1a, w1b, _b2(b1), w2, _b2(b2), w3, _b2(b3))

    tw, tb = params["task"]
    twp = jnp.concatenate([tw, jnp.zeros((6, 128), f32)], axis=0)
    ts = jnp.concatenate([task_sp.astype(f32),
                          jnp.zeros((_B, 6), f32)], axis=1).reshape(_B, 1, 8)

    (g1, gb1), (g2, gb2), (g3, gb3) = params["gsa"]
    g1a = g1[:256]
    g1b = jnp.concatenate([g1[256:259], jnp.zeros((5, 256), f32)], axis=0)
    (f1, fb1), (f2, fb2) = params["fp3"]
    f1a = f1[:1024]
    f1b = f1[1024:1280]
    (a1, ab1), (a2, ab2) = params["fp2"]
    a1x = a1[:256]
    a1y = a1[256:384]
    (p1, pb1), (p2, pb2), (p3, pb3) = params["fp1"]
    l1, lb1 = params["lin1"]
    l2, lb2 = params["lin2"]
    l3, lb3 = params["lin3"]
    l3p = jnp.concatenate([l3, jnp.zeros((128, 5), f32)], axis=1)
    lb3p = jnp.concatenate([_b2(lb3), jnp.zeros((1, 5), f32)], axis=1)

    weights = [twp, _b2(tb),
               g1a, g1b, _b2(gb1), g2, _b2(gb2), g3, _b2(gb3),
               f1a, f1b, _b2(fb1), f2, _b2(fb2),
               a1x, a1y, _b2(ab1), a2, _b2(ab2),
               p1, _b2(pb1), p2, _b2(pb2), p3, _b2(pb3),
               l1, _b2(lb1), l2, _b2(lb2), l3p, lb3p]

    out = pl.pallas_call(
        _tail_kernel,
        grid=(_B,),
        compiler_params=pltpu.CompilerParams(dimension_semantics=("parallel",)),
        in_specs=[bspec((_S2P, 256)), bspec((_S2P, 8)), bspec((8, _S2P)),
                  bspec((_S1P, 8)), bspec((8, _S1P)), bspec((_S1P, 128)),
                  bspec((_N, 8)), bspec((1, 8))]
                 + [wspec2(w) for w in weights],
        out_specs=bspec((_N, 8)),
        out_shape=jax.ShapeDtypeStruct((_B, _N, 8), f32),
    )(x2, pos2, pos2T, pos1, pos1T, x1, pos8, ts, *weights)

    return out[:, :, :3].reshape(_B * _N, 3)


# speedup vs baseline: 1.4390x; 1.4390x over previous
import jax
import jax.numpy as jnp
from jax.experimental import pallas as pl
from jax.experimental.pallas import tpu as pltpu

_B = 16
_N = 2048
_S1 = 410
_S1P = 416
_S2 = 103
_S2P = 104
_R1 = 0.2
_R2 = 0.4
_K = 64
_BIG = 1e30
_PADC = 1e5


def _fps_kernel(posL_ref, pos1L_ref, pos2L_ref, d1_ref, d2_ref):
    pos1L_ref[...] = jnp.full((3, _S1P, _B), _PADC, jnp.float32)
    pos2L_ref[...] = jnp.full((3, _S2P, _B), _PADC, jnp.float32)
    d0 = jnp.zeros((_N, _B), jnp.float32)
    for l in range(3):
        pos1L_ref[l, 0, :] = posL_ref[l, 0, :]
        df = posL_ref[l] - posL_ref[l, 0:1, :]
        d0 = d0 + df * df
    d1_ref[...] = d0

    def body1(i, carry):
        d = d1_ref[...]
        nxt = jnp.argmax(d, axis=0)
        onehot = jax.lax.broadcasted_iota(jnp.int32, (_N, _B), 0) == nxt[None, :]
        dn = jnp.zeros((_N, _B), jnp.float32)
        for l in range(3):
            c = jnp.max(jnp.where(onehot, posL_ref[l], -1.0), axis=0)
            pos1L_ref[l, i, :] = c
            df = posL_ref[l] - c[None, :]
            dn = dn + df * df
        d1_ref[...] = jnp.minimum(d, dn)
        return carry

    jax.lax.fori_loop(1, _S1, body1, 0)

    d0b = jnp.zeros((_S1P, _B), jnp.float32)
    for l in range(3):
        pos2L_ref[l, 0, :] = pos1L_ref[l, 0, :]
        df = pos1L_ref[l] - pos1L_ref[l, 0:1, :]
        d0b = d0b + df * df
    rows = jax.lax.broadcasted_iota(jnp.int32, (_S1P, _B), 0)
    d2_ref[...] = jnp.where(rows < _S1, d0b, -_BIG)

    def body2(i, carry):
        d = d2_ref[...]
        nxt = jnp.argmax(d, axis=0)
        onehot = jax.lax.broadcasted_iota(jnp.int32, (_S1P, _B), 0) == nxt[None, :]
        dn = jnp.zeros((_S1P, _B), jnp.float32)
        for l in range(3):
            c = jnp.max(jnp.where(onehot, pos1L_ref[l], -1.0), axis=0)
            pos2L_ref[l, i, :] = c
            df = pos1L_ref[l] - c[None, :]
            dn = dn + df * df
        d2_ref[...] = jnp.minimum(d, jnp.where(rows < _S1, dn, -_BIG))
        return carry

    jax.lax.fori_loop(1, _S2, body2, 0)


def _sa1_kernel(pos1_ref, posT_ref, w1_ref, b1_ref, w2_ref, b2_ref,
                w3_ref, b3_ref, x1_ref, d2_ref):
    pos1 = pos1_ref[0]
    posT = posT_ref[0]
    d2 = jnp.zeros((_S1P, _N), jnp.float32)
    for l in range(3):
        df = pos1[:, l:l + 1] - posT[l:l + 1, :]
        d2 = d2 + df * df
    d2_ref[...] = jnp.where(d2 <= _R1 * _R1, d2, _BIG)
    x1_ref[0] = jnp.full((_S1P, 128), -1.0, jnp.float32)

    dot = lambda a, b: jnp.dot(a, b, preferred_element_type=jnp.float32,
                               precision=jax.lax.Precision.DEFAULT)

    def body(i, carry):
        d = d2_ref[...]
        idx = jnp.argmin(d, axis=1)
        m2 = jnp.min(d, axis=1, keepdims=True)
        onehot = jax.lax.broadcasted_iota(jnp.int32, (_S1P, _N), 1) == idx[:, None]
        d2_ref[...] = jnp.where(onehot, _BIG, d)
        cols = []
        for l in range(3):
            c = jnp.max(jnp.where(onehot, posT[l:l + 1, :], -1.0), axis=1)
            cols.append(c[:, None])
        cols.append(jnp.zeros((_S1P, 5), jnp.float32))
        rel = jnp.concatenate(cols, axis=1) - pos1
        h = jax.nn.relu(dot(rel, w1_ref[...]) + b1_ref[...])
        h = jax.nn.relu(dot(h, w2_ref[...]) + b2_ref[...])
        h = jax.nn.relu(dot(h, w3_ref[...]) + b3_ref[...])
        h = jnp.where(m2 < _BIG * 0.5, h, -1.0)
        x1_ref[0] = jnp.maximum(x1_ref[0], h)
        return carry

    jax.lax.fori_loop(0, _K, body, 0)


def _sa2_kernel(pos2_ref, pos1T_ref, x1_ref, w1a_ref, w1b_ref, b1_ref,
                w2_ref, b2_ref, w3_ref, b3_ref, x2_ref, d2_ref):
    pos2 = pos2_ref[0]
    pos1T = pos1T_ref[0]
    x1 = x1_ref[0]
    d2 = jnp.zeros((_S2P, _S1P), jnp.float32)
    for l in range(3):
        df = pos2[:, l:l + 1] - pos1T[l:l + 1, :]
        d2 = d2 + df * df
    d2_ref[...] = jnp.where(d2 <= _R2 * _R2, d2, _BIG)
    x2_ref[0] = jnp.full((_S2P, 256), -1.0, jnp.float32)

    dot = lambda a, b: jnp.dot(a, b, preferred_element_type=jnp.float32,
                               precision=jax.lax.Precision.DEFAULT)

    def body(i, carry):
        d = d2_ref[...]
        idx = jnp.argmin(d, axis=1)
        m2 = jnp.min(d, axis=1, keepdims=True)
        onehot = jax.lax.broadcasted_iota(jnp.int32, (_S2P, _S1P), 1) == idx[:, None]
        d2_ref[...] = jnp.where(onehot, _BIG, d)
        cols = []
        for l in range(3):
            c = jnp.max(jnp.where(onehot, pos1T[l:l + 1, :], -1.0), axis=1)
            cols.append(c[:, None])
        cols.append(jnp.zeros((_S2P, 5), jnp.float32))
        rel = jnp.concatenate(cols, axis=1) - pos2
        xg = dot(onehot.astype(jnp.float32), x1)
        h = jax.nn.relu(dot(xg, w1a_ref[...]) + dot(rel, w1b_ref[...]) + b1_ref[...])
        h = jax.nn.relu(dot(h, w2_ref[...]) + b2_ref[...])
        h = jax.nn.relu(dot(h, w3_ref[...]) + b3_ref[...])
        h = jnp.where(m2 < _BIG * 0.5, h, -1.0)
        x2_ref[0] = jnp.maximum(x2_ref[0], h)
        return carry

    jax.lax.fori_loop(0, _K, body, 0)


def _knn_wmat(dst, srcT, nd, ns):
    d2 = jnp.zeros((nd, ns), jnp.float32)
    for l in range(3):
        df = dst[:, l:l + 1] - srcT[l:l + 1, :]
        d2 = d2 + df * df
    wm = jnp.zeros((nd, ns), jnp.float32)
    wsum = jnp.zeros((nd, 1), jnp.float32)
    for _ in range(3):
        idx = jnp.argmin(d2, axis=1)
        m = jnp.min(d2, axis=1)
        onehot = jax.lax.broadcasted_iota(jnp.int32, (nd, ns), 1) == idx[:, None]
        w = 1.0 / jnp.maximum(m, 1e-16)
        wm = wm + jnp.where(onehot, w[:, None], 0.0)
        wsum = wsum + w[:, None]
        d2 = jnp.where(onehot, _BIG, d2)
    return wm / wsum


def _tail_kernel(x2_ref, pos2_ref, pos2T_ref, pos1_ref, pos1T_ref, x1_ref,
                 pos_ref, ts_ref,
                 tw_ref, tb_ref,
                 g1a_ref, g1b_ref, gb1_ref, g2_ref, gb2_ref, g3_ref, gb3_ref,
                 f1a_ref, f1b_ref, fb1_ref, f2_ref, fb2_ref,
                 a1x_ref, a1y_ref, ab1_ref, a2_ref, ab2_ref,
                 p1_ref, pb1_ref, p2_ref, pb2_ref, p3_ref, pb3_ref,
                 l1_ref, lb1_ref, l2_ref, lb2_ref, l3_ref, lb3_ref,
                 out_ref):
    x2 = x2_ref[0]
    pos2 = pos2_ref[0]
    pos2T = pos2T_ref[0]
    pos1 = pos1_ref[0]
    pos1T = pos1T_ref[0]
    x1 = x1_ref[0]
    posb = pos_ref[0]

    dot = lambda a, b: jnp.dot(a, b, preferred_element_type=jnp.float32,
                               precision=jax.lax.Precision.DEFAULT)

    t = jax.nn.relu(dot(ts_ref[0], tw_ref[...]) + tb_ref[...])

    h = jax.nn.relu(dot(x2, g1a_ref[...]) + dot(pos2, g1b_ref[...]) + gb1_ref[...])
    h = jax.nn.relu(dot(h, g2_ref[...]) + gb2_ref[...])
    h = jax.nn.relu(dot(h, g3_ref[...]) + gb3_ref[...])
    rows = jax.lax.broadcasted_iota(jnp.int32, (_S2P, 1024), 0)
    h = jnp.where(rows < _S2, h, -_BIG)
    x3 = jnp.max(h, axis=0, keepdims=True)
    t8 = jnp.concatenate([t] * 8, axis=1)
    x3 = x3 * t8

    fh = jax.nn.relu(dot(x3, f1a_ref[...]) + dot(x2, f1b_ref[...]) + fb1_ref[...])
    fp3_x = jax.nn.relu(dot(fh, f2_ref[...]) + fb2_ref[...])
    t2 = jnp.concatenate([t] * 2, axis=1)
    fp3_x = fp3_x * t2

    wm2 = _knn_wmat(pos1, pos2T, _S1P, _S2P)
    fp2_i = dot(wm2, fp3_x)
    fh = jax.nn.relu(dot(fp2_i, a1x_ref[...]) + dot(x1, a1y_ref[...]) + ab1_ref[...])
    fp2_x = jax.nn.relu(dot(fh, a2_ref[...]) + ab2_ref[...])
    fp2_x = fp2_x * t

    wm1 = _knn_wmat(posb, pos1T, _N, _S1P)
    fp1_i = dot(wm1, fp2_x)
    fh = jax.nn.relu(dot(fp1_i, p1_ref[...]) + pb1_ref[...])
    fh = jax.nn.relu(dot(fh, p2_ref[...]) + pb2_ref[...])
    fh = jax.nn.relu(dot(fh, p3_ref[...]) + pb3_ref[...])

    fh = jax.nn.relu(dot(fh, l1_ref[...]) + lb1_ref[...])
    fh = jax.nn.relu(dot(fh, l2_ref[...]) + lb2_ref[...])
    out_ref[0] = dot(fh, l3_ref[...]) + lb3_ref[...]


def _b2(b):
    return b.reshape(1, -1)


def kernel(pos, batch, task_sp, params):
    f32 = jnp.float32
    posb = pos.reshape(_B, _N, 3).astype(f32)
    pos8 = jnp.concatenate([posb, jnp.zeros((_B, _N, 5), f32)], axis=2)
    posT = jnp.concatenate([jnp.transpose(posb, (0, 2, 1)),
                            jnp.zeros((_B, 5, _N), f32)], axis=1)
    posL = jnp.transpose(posb, (2, 1, 0))

    pos1L, pos2L = pl.pallas_call(
        _fps_kernel,
        out_shape=[jax.ShapeDtypeStruct((3, _S1P, _B), f32),
                   jax.ShapeDtypeStruct((3, _S2P, _B), f32)],
        scratch_shapes=[pltpu.VMEM((_N, _B), f32),
                        pltpu.VMEM((_S1P, _B), f32)],
    )(posL)

    pos1b = jnp.transpose(pos1L, (2, 1, 0))
    pos2b = jnp.transpose(pos2L, (2, 1, 0))
    pos1 = jnp.concatenate([pos1b, jnp.zeros((_B, _S1P, 5), f32)], axis=2)
    pos2 = jnp.concatenate([pos2b, jnp.zeros((_B, _S2P, 5), f32)], axis=2)
    pos1T = jnp.concatenate([jnp.transpose(pos1b, (0, 2, 1)),
                             jnp.zeros((_B, 5, _S1P), f32)], axis=1)
    pos2T = jnp.concatenate([jnp.transpose(pos2b, (0, 2, 1)),
                             jnp.zeros((_B, 5, _S2P), f32)], axis=1)

    (w1, b1), (w2, b2), (w3, b3) = params["sa1"]
    w1p = jnp.concatenate([w1, jnp.zeros((5, 64), f32)], axis=0)

    def bspec(shape):
        return pl.BlockSpec((1,) + shape, lambda i: (i, 0, 0))

    def wspec2(a):
        return pl.BlockSpec(a.shape, lambda i: (0,) * a.ndim)

    x1 = pl.pallas_call(
        _sa1_kernel,
        grid=(_B,),
        compiler_params=pltpu.CompilerParams(dimension_semantics=("parallel",)),
        in_specs=[bspec((_S1P, 8)), bspec((8, _N)),
                  wspec2(w1p), wspec2(_b2(b1)), wspec2(w2), wspec2(_b2(b2)),
                  wspec2(w3), wspec2(_b2(b3))],
        out_specs=bspec((_S1P, 128)),
        out_shape=jax.ShapeDtypeStruct((_B, _S1P, 128), f32),
        scratch_shapes=[pltpu.VMEM((_S1P, _N), f32)],
    )(pos1, posT, w1p, _b2(b1), w2, _b2(b2), w3, _b2(b3))

    (w1, b1), (w2, b2), (w3, b3) = params["sa2"]
    w1a = w1[:128]
    w1b = jnp.concatenate([w1[128:131], jnp.zeros((5, 128), f32)], axis=0)
    x2 = pl.pallas_call(
        _sa2_kernel,
        grid=(_B,),
        compiler_params=pltpu.CompilerParams(dimension_semantics=("parallel",)),
        in_specs=[bspec((_S2P, 8)), bspec((8, _S1P)), bspec((_S1P, 128)),
                  wspec2(w1a), wspec2(w1b), wspec2(_b2(b1)),
                  wspec2(w2), wspec2(_b2(b2)), wspec2(w3), wspec2(_b2(b3))],
        out_specs=bspec((_S2P, 256)),
        out_shape=jax.ShapeDtypeStruct((_B, _S2P, 256), f32),
        scratch_shapes=[pltpu.VMEM((_S2P, _S1P), f32)],
    )(pos2, pos1T, x1, w1a, w1b, _b2(b1), w2, _b2(b2), w3, _b2(b3))

    tw, tb = params["task"]
    twp = jnp.concatenate([tw, jnp.zeros((6, 128), f32)], axis=0)
    ts = jnp.concatenate([task_sp.astype(f32),
                          jnp.zeros((_B, 6), f32)], axis=1).reshape(_B, 1, 8)

    (g1, gb1), (g2, gb2), (g3, gb3) = params["gsa"]
    g1a = g1[:256]
    g1b = jnp.concatenate([g1[256:259], jnp.zeros((5, 256), f32)], axis=0)
    (f1, fb1), (f2, fb2) = params["fp3"]
    f1a = f1[:1024]
    f1b = f1[1024:1280]
    (a1, ab1), (a2, ab2) = params["fp2"]
    a1x = a1[:256]
    a1y = a1[256:384]
    (p1, pb1), (p2, pb2), (p3, pb3) = params["fp1"]
    l1, lb1 = params["lin1"]
    l2, lb2 = params["lin2"]
    l3, lb3 = params["lin3"]
    l3p = jnp.concatenate([l3, jnp.zeros((128, 5), f32)], axis=1)
    lb3p = jnp.concatenate([_b2(lb3), jnp.zeros((1, 5), f32)], axis=1)

    weights = [twp, _b2(tb),
               g1a, g1b, _b2(gb1), g2, _b2(gb2), g3, _b2(gb3),
               f1a, f1b, _b2(fb1), f2, _b2(fb2),
               a1x, a1y, _b2(ab1), a2, _b2(ab2),
               p1, _b2(pb1), p2, _b2(pb2), p3, _b2(pb3),
               l1, _b2(lb1), l2, _b2(lb2), l3p, lb3p]

    out = pl.pallas_call(
        _tail_kernel,
        grid=(_B,),
        compiler_params=pltpu.CompilerParams(dimension_semantics=("parallel",)),
        in_specs=[bspec((_S2P, 256)), bspec((_S2P, 8)), bspec((8, _S2P)),
                  bspec((_S1P, 8)), bspec((8, _S1P)), bspec((_S1P, 128)),
                  bspec((_N, 8)), bspec((1, 8))]
                 + [wspec2(w) for w in weights],
        out_specs=bspec((_N, 8)),
        out_shape=jax.ShapeDtypeStruct((_B, _N, 8), f32),
    )(x2, pos2, pos2T, pos1, pos1T, x1, pos8, ts, *weights)

    return out[:, :, :3].reshape(_B * _N, 3)
